# baseline (speedup 1.0000x reference)
import jax
import jax.numpy as jnp
from jax import lax
from jax.experimental import pallas as pl
from jax.experimental.pallas import tpu as pltpu
from jax.experimental.pallas import tpu_sc as plsc

N = 50000
E = 800000
B = 256
H = 64
V = 51

NC = 2
NS = 16
CE = 128

N_PAD = 51200
MQ = N_PAD * 32 // 128
NODE_T = N_PAD // NS
EDGE_C = 400
E_T = EDGE_C * CE
E_PAD = NS * E_T
AGG_ROWS = N_PAD + CE
ZERO_T = AGG_ROWS // NS
PB = B + 8
RB = 512

_MESH = plsc.VectorSubcoreMesh(core_axis_name="c", subcore_axis_name="s")
_SC_PARAMS = pltpu.CompilerParams(use_tc_tiling_on_sc=False)


def _embed_body(x2, table, o0, o1, tbl_s, idx_v, rows_v, semi, semg, sems):
    c = lax.axis_index("c")
    s = lax.axis_index("s")
    nbase = s * NODE_T
    NCH = NODE_T // CE

    @pl.when(s == 0)
    def _ld():
        pltpu.sync_copy(table, tbl_s)

    plsc.subcore_barrier()

    def fire_idx(t, slot):
        pltpu.async_copy(x2.at[pl.ds(c * N_PAD + nbase + t * CE, CE)],
                         idx_v.at[slot], semi.at[slot])

    def wait_idx(slot):
        pltpu.make_async_copy(x2.at[pl.ds(0, CE)], idx_v.at[slot],
                              semi.at[slot]).wait()

    def fire_gather(slot):
        pltpu.async_copy(tbl_s.at[idx_v.at[slot]], rows_v.at[slot],
                         semg.at[slot])

    def wait_gather(slot):
        pltpu.make_async_copy(table.at[pl.ds(0, CE)], rows_v.at[slot],
                              semg.at[slot]).wait()

    def fire_store(t, slot):
        @pl.when(c == 0)
        def _s0():
            pltpu.async_copy(rows_v.at[slot],
                             o0.at[pl.ds(nbase + t * CE, CE)], sems.at[slot])

        @pl.when(c == 1)
        def _s1():
            pltpu.async_copy(rows_v.at[slot],
                             o1.at[pl.ds(nbase + t * CE, CE)], sems.at[slot])

    def wait_store(slot):
        pltpu.make_async_copy(rows_v.at[slot], o0.at[pl.ds(0, CE)],
                              sems.at[slot]).wait()

    fire_idx(0, 0)
    fire_idx(1, 1)
    wait_idx(0)
    fire_gather(0)

    def step(jj, carry):
        for b in range(5):
            i = jj * 5 + b

            @pl.when(i >= 2)
            def _fs(b=b):
                wait_store((b + 3) % 5)

            @pl.when(i + 2 < NCH)
            def _fi(i=i, b=b):
                fire_idx(i + 2, (b + 2) % 5)

            @pl.when(i + 1 < NCH)
            def _fg(b=b):
                wait_idx((b + 1) % 5)
                fire_gather((b + 1) % 5)

            wait_gather(b)
            fire_store(i, b)
        return carry

    lax.fori_loop(0, NCH // 5, step, 0)
    wait_store((NCH - 2) % 5)
    wait_store((NCH - 1) % 5)


def _embed(x2, table):
    return pl.kernel(
        _embed_body,
        out_type=(jax.ShapeDtypeStruct((N_PAD, 32), jnp.float32),
                  jax.ShapeDtypeStruct((N_PAD, 32), jnp.float32)),
        mesh=_MESH,
        compiler_params=_SC_PARAMS,
        scratch_types=[
            pltpu.VMEM_SHARED((2 * V, 32), jnp.float32),
            pltpu.VMEM((5, CE), jnp.int32),
            pltpu.VMEM((5, CE, 32), jnp.float32),
            pltpu.SemaphoreType.DMA((5,)),
            pltpu.SemaphoreType.DMA((5,)),
            pltpu.SemaphoreType.DMA((5,)),
        ],
    )(x2, table)


def _agg_body(h0, h1, src1, dst1, zrows, o0, o1, agg_s,
              sidx, didx, rows, semi, semg, sems):
    c = lax.axis_index("c")
    s = lax.axis_index("s")
    pltpu.sync_copy(zrows.at[pl.ds(s * ZERO_T, ZERO_T)],
                    agg_s.at[pl.ds(s * ZERO_T, ZERO_T)])
    plsc.subcore_barrier()

    ebase = s * E_T

    def fire_idx(t, slot):
        pltpu.async_copy(src1.at[pl.ds(ebase + t * CE, CE)], sidx.at[slot],
                         semi.at[slot])
        pltpu.async_copy(dst1.at[pl.ds(ebase + t * CE, CE)], didx.at[slot],
                         semi.at[slot])

    def wait_idx(slot):
        pltpu.make_async_copy(dst1.at[pl.ds(0, CE)], sidx.at[slot],
                              semi.at[slot]).wait()
        pltpu.make_async_copy(dst1.at[pl.ds(0, CE)], didx.at[slot],
                              semi.at[slot]).wait()

    def fire_gather(islot, rslot):
        @pl.when(c == 0)
        def _g0():
            pltpu.async_copy(h0.at[sidx.at[islot]], rows.at[rslot],
                             semg.at[rslot])

        @pl.when(c == 1)
        def _g1():
            pltpu.async_copy(h1.at[sidx.at[islot]], rows.at[rslot],
                             semg.at[rslot])

    def wait_gather(rslot):
        pltpu.make_async_copy(h0.at[pl.ds(0, CE)], rows.at[rslot],
                              semg.at[rslot]).wait()

    def fire_scatter(islot, rslot):
        pltpu.async_copy(rows.at[rslot], agg_s.at[didx.at[islot]],
                         sems.at[rslot], add=True)

    def wait_scatter(rslot):
        pltpu.make_async_copy(h0.at[pl.ds(0, CE)], rows.at[rslot],
                              sems.at[rslot]).wait()

    for t in range(5):
        fire_idx(t, t)
    for t in range(3):
        wait_idx(t)
        fire_gather(t, t)

    def step(jj, carry):
        for b in range(10):
            i = jj * 10 + b

            @pl.when(i >= 2)
            def _free(b=b):
                wait_scatter((b + 3) % 5)

            @pl.when(i + 5 < EDGE_C)
            def _idx(i=i, b=b):
                fire_idx(i + 5, (b + 5) % 10)

            @pl.when(i + 3 < EDGE_C)
            def _gath(b=b):
                wait_idx((b + 3) % 10)
                fire_gather((b + 3) % 10, (b + 3) % 5)

            wait_gather(b % 5)
            fire_scatter(b % 10, b % 5)
        return carry

    lax.fori_loop(0, EDGE_C // 10, step, 0)
    for t in range(EDGE_C - 2, EDGE_C):
        wait_scatter(t % 5)
    plsc.subcore_barrier()

    @pl.when(c == 0)
    def _o0():
        pltpu.sync_copy(agg_s.at[pl.ds(s * NODE_T, NODE_T)],
                        o0.at[pl.ds(s * NODE_T, NODE_T)])

    @pl.when(c == 1)
    def _o1():
        pltpu.sync_copy(agg_s.at[pl.ds(s * NODE_T, NODE_T)],
                        o1.at[pl.ds(s * NODE_T, NODE_T)])


def _agg(h0, h1, src1, dst1, zrows):
    return pl.kernel(
        _agg_body,
        out_type=(jax.ShapeDtypeStruct((N_PAD, 32), jnp.float32),
                  jax.ShapeDtypeStruct((N_PAD, 32), jnp.float32)),
        mesh=_MESH,
        compiler_params=_SC_PARAMS,
        scratch_types=[
            pltpu.VMEM_SHARED((AGG_ROWS, 32), jnp.float32),
            pltpu.VMEM((10, CE), jnp.int32),
            pltpu.VMEM((10, CE), jnp.int32),
            pltpu.VMEM((5, CE, 32), jnp.float32),
            pltpu.SemaphoreType.DMA((10,)),
            pltpu.SemaphoreType.DMA((5,)),
            pltpu.SemaphoreType.DMA((5,)),
        ],
    )(h0, h1, src1, dst1, zrows)


def _pool_body(h0, h1, batchp, ones_h, zpool, zcnt, outp, outc,
               pool_s, cnt_s, bidx, rows_v, ones_v, semr, semb):
    c = lax.axis_index("c")
    s = lax.axis_index("s")
    nbase = s * NODE_T
    NCH = NODE_T // CE

    @pl.when(s == 0)
    def _z():
        pltpu.sync_copy(zpool, pool_s)
        pltpu.sync_copy(zcnt, cnt_s)

    pltpu.sync_copy(ones_h, ones_v)
    plsc.subcore_barrier()

    def fire_read(t, slot):
        @pl.when(c == 0)
        def _r0():
            pltpu.async_copy(h0.at[pl.ds(nbase + t * CE, CE)],
                             rows_v.at[slot], semr.at[slot])

        @pl.when(c == 1)
        def _r1():
            pltpu.async_copy(h1.at[pl.ds(nbase + t * CE, CE)],
                             rows_v.at[slot], semr.at[slot])

        pltpu.async_copy(batchp.at[pl.ds(nbase + t * CE, CE)],
                         bidx.at[slot], semb.at[slot])

    def wait_read(slot):
        pltpu.make_async_copy(h0.at[pl.ds(0, CE)], rows_v.at[slot],
                              semr.at[slot]).wait()
        pltpu.make_async_copy(batchp.at[pl.ds(0, CE)], bidx.at[slot],
                              semb.at[slot]).wait()

    fire_read(0, 0)
    fire_read(1, 1)

    def step(jj, carry):
        for b in range(5):
            i = jj * 5 + b

            @pl.when(i + 2 < NCH)
            def _fr(i=i, b=b):
                fire_read(i + 2, (b + 2) % 5)

            wait_read(b)
            pltpu.sync_copy(rows_v.at[b], pool_s.at[bidx.at[b]], add=True)
            pltpu.sync_copy(ones_v, cnt_s.at[bidx.at[b]], add=True)
        return carry

    lax.fori_loop(0, NCH // 5, step, 0)
    plsc.subcore_barrier()

    @pl.when(s == 0)
    def _out():
        pltpu.sync_copy(pool_s, outp.at[c])
        pltpu.sync_copy(cnt_s, outc.at[c])


def _pool(h0, h1, batchp, ones_h, zpool, zcnt):
    return pl.kernel(
        _pool_body,
        out_type=(jax.ShapeDtypeStruct((NC, PB, 32), jnp.float32),
                  jax.ShapeDtypeStruct((NC, PB, 16), jnp.float32)),
        mesh=_MESH,
        compiler_params=_SC_PARAMS,
        scratch_types=[
            pltpu.VMEM_SHARED((PB, 32), jnp.float32),
            pltpu.VMEM_SHARED((PB, 16), jnp.float32),
            pltpu.VMEM((5, CE), jnp.int32),
            pltpu.VMEM((5, CE, 32), jnp.float32),
            pltpu.VMEM((CE, 16), jnp.float32),
            pltpu.SemaphoreType.DMA((5,)),
            pltpu.SemaphoreType.DMA((5,)),
        ],
    )(h0, h1, batchp, ones_h, zpool, zcnt)


def _mlp_body(eps_ref, h0_ref, h1_ref, a0_ref, a1_ref, w1a_ref, w1b_ref,
              b1_ref, w20_ref, w21_ref, b20_ref, b21_ref, o0_ref, o1_ref):
    scale = 1.0 + eps_ref[0]
    z0 = scale * h0_ref[...] + a0_ref[...]
    z1 = scale * h1_ref[...] + a1_ref[...]
    t = jnp.dot(z0, w1a_ref[...], preferred_element_type=jnp.float32)
    t += jnp.dot(z1, w1b_ref[...], preferred_element_type=jnp.float32)
    t = jnp.maximum(t + b1_ref[...], 0.0)
    y0 = jnp.dot(t, w20_ref[...], preferred_element_type=jnp.float32)
    y1 = jnp.dot(t, w21_ref[...], preferred_element_type=jnp.float32)
    o0_ref[...] = jnp.maximum(y0 + b20_ref[...], 0.0)
    o1_ref[...] = jnp.maximum(y1 + b21_ref[...], 0.0)


def _mlp_layer(eps_l, h0, h1, a0, a1, w1, b1, w2, b2):
    h0q = h0.reshape(MQ, 128)
    h1q = h1.reshape(MQ, 128)
    a0q = a0.reshape(MQ, 128)
    a1q = a1.reshape(MQ, 128)
    eye4 = jnp.eye(4, dtype=jnp.float32)
    w1a = jnp.kron(eye4, w1[:32, :])
    w1b = jnp.kron(eye4, w1[32:, :])
    b1big = jnp.tile(b1, 4)
    w20 = jnp.kron(eye4, w2[:, :32])
    w21 = jnp.kron(eye4, w2[:, 32:])
    b20 = jnp.tile(b2[:32], 4)
    b21 = jnp.tile(b2[32:], 4)
    grid = (MQ // RB,)
    blk = lambda i: (i, 0)
    fixed = lambda i: (0, 0)
    vec = lambda i: (0,)
    o0q, o1q = pl.pallas_call(
        _mlp_body,
        grid=grid,
        in_specs=[
            pl.BlockSpec(memory_space=pltpu.SMEM),
            pl.BlockSpec((RB, 128), blk),
            pl.BlockSpec((RB, 128), blk),
            pl.BlockSpec((RB, 128), blk),
            pl.BlockSpec((RB, 128), blk),
            pl.BlockSpec((128, 256), fixed),
            pl.BlockSpec((128, 256), fixed),
            pl.BlockSpec((256,), vec),
            pl.BlockSpec((256, 128), fixed),
            pl.BlockSpec((256, 128), fixed),
            pl.BlockSpec((128,), vec),
            pl.BlockSpec((128,), vec),
        ],
        out_specs=[pl.BlockSpec((RB, 128), blk), pl.BlockSpec((RB, 128), blk)],
        out_shape=[jax.ShapeDtypeStruct((MQ, 128), jnp.float32),
                   jax.ShapeDtypeStruct((MQ, 128), jnp.float32)],
    )(eps_l.reshape(1), h0q, h1q, a0q, a1q, w1a, w1b, b1big, w20, w21,
      b20, b21)
    return o0q.reshape(N_PAD, 32), o1q.reshape(N_PAD, 32)


def _head_body(p0_ref, p1_ref, cnt_ref, w1_ref, b1_ref, w2f_ref, sel_ref,
               b2_ref, out_ref):
    cnt = jnp.maximum(cnt_ref[...][:, 0:1], 1.0)
    g = jnp.concatenate([p0_ref[...], p1_ref[...]], axis=1) / cnt
    t = jnp.maximum(jnp.dot(g, w1_ref[...], preferred_element_type=jnp.float32)
                    + b1_ref[...], 0.0)
    u = t * w2f_ref[...]
    out_ref[...] = jnp.dot(u, sel_ref[...],
                           preferred_element_type=jnp.float32) + b2_ref[...]


def _head(p0, p1, cnt, w1cat, b1cat, w2flat, sel, b2row):
    P = sel.shape[1]
    return pl.pallas_call(
        _head_body,
        out_shape=jax.ShapeDtypeStruct((B, P), jnp.float32),
    )(p0, p1, cnt, w1cat, b1cat, w2flat, sel, b2row)


def kernel(x, edge_index, batch, embed, CW1, Cb1, CW2, Cb2, eps, HW1, Hb1,
           HW2, Hb2):
    x = x.astype(jnp.int32)
    src = edge_index[0].astype(jnp.int32)
    dst = edge_index[1].astype(jnp.int32)
    batch = batch.astype(jnp.int32)

    x_pad = jnp.pad(x, (0, N_PAD - N))
    x2 = jnp.concatenate([x_pad, x_pad + V])
    table = jnp.concatenate([embed[:, :32], embed[:, 32:]], axis=0)

    src1 = jnp.pad(src, (0, E_PAD - E))
    dst1 = jnp.pad(dst, (0, E_PAD - E), constant_values=N_PAD)
    zrows = jnp.zeros((AGG_ROWS, 32), jnp.float32)

    batchp = jnp.pad(batch, (0, N_PAD - N), constant_values=B)
    ones_h = jnp.ones((CE, 16), jnp.float32)
    zpool = jnp.zeros((PB, 32), jnp.float32)
    zcnt = jnp.zeros((PB, 16), jnp.float32)

    h0, h1 = _embed(x2, table)
    for l in range(3):
        a0, a1 = _agg(h0, h1, src1, dst1, zrows)
        h0, h1 = _mlp_layer(eps[l], h0, h1, a0, a1, CW1[l], Cb1[l], CW2[l],
                            Cb2[l])

    pool, cnt = _pool(h0, h1, batchp, ones_h, zpool, zcnt)

    P = HW1.shape[0]
    w1cat = HW1.transpose(1, 0, 2).reshape(H, P * H)
    b1cat = Hb1.reshape(P * H)
    w2flat = HW2[:, :, 0].reshape(P * H)
    sel = jnp.repeat(jnp.eye(P, dtype=jnp.float32), H, axis=0)
    b2row = Hb2[:, 0]
    return _head(pool[0, :B], pool[1, :B], cnt[0, :B], w1cat, b1cat, w2flat,
                 sel, b2row)


# speedup vs baseline: 1.8172x; 1.8172x over previous
import jax
import jax.numpy as jnp
from jax import lax
from jax.experimental import pallas as pl
from jax.experimental.pallas import tpu as pltpu
from jax.experimental.pallas import tpu_sc as plsc

N = 50000
E = 800000
B = 256
H = 64
V = 51

NC = 2
NS = 16
CE = 128

N_PAD = 51200
MQ = N_PAD * 32 // 128
NODE_T = N_PAD // NS
EDGE_C = 392
E_T = EDGE_C * CE
E_PAD = NS * E_T
AGG_ROWS = N_PAD + CE
ZERO_T = AGG_ROWS // NS
PB = B + 8
RB = 512

_MESH = plsc.VectorSubcoreMesh(core_axis_name="c", subcore_axis_name="s")
_SC_PARAMS = pltpu.CompilerParams(use_tc_tiling_on_sc=False)


def _embed_body(x2, table, o0, o1, tbl_s, idx_v, rows_v, semi, semg, sems):
    c = lax.axis_index("c")
    s = lax.axis_index("s")
    nbase = s * NODE_T
    NCH = NODE_T // CE

    @pl.when(s == 0)
    def _ld():
        pltpu.sync_copy(table, tbl_s)

    plsc.subcore_barrier()

    def fire_idx(t, slot):
        pltpu.async_copy(x2.at[pl.ds(c * N_PAD + nbase + t * CE, CE)],
                         idx_v.at[slot], semi.at[slot])

    def wait_idx(slot):
        pltpu.make_async_copy(x2.at[pl.ds(0, CE)], idx_v.at[slot],
                              semi.at[slot]).wait()

    def fire_gather(slot):
        pltpu.async_copy(tbl_s.at[idx_v.at[slot]], rows_v.at[slot],
                         semg.at[slot])

    def wait_gather(slot):
        pltpu.make_async_copy(table.at[pl.ds(0, CE)], rows_v.at[slot],
                              semg.at[slot]).wait()

    def fire_store(t, slot):
        @pl.when(c == 0)
        def _s0():
            pltpu.async_copy(rows_v.at[slot],
                             o0.at[pl.ds(nbase + t * CE, CE)], sems.at[slot])

        @pl.when(c == 1)
        def _s1():
            pltpu.async_copy(rows_v.at[slot],
                             o1.at[pl.ds(nbase + t * CE, CE)], sems.at[slot])

    def wait_store(slot):
        pltpu.make_async_copy(rows_v.at[slot], o0.at[pl.ds(0, CE)],
                              sems.at[slot]).wait()

    fire_idx(0, 0)
    fire_idx(1, 1)
    wait_idx(0)
    fire_gather(0)

    def step(jj, carry):
        for b in range(5):
            i = jj * 5 + b

            @pl.when(i >= 2)
            def _fs(b=b):
                wait_store((b + 3) % 5)

            @pl.when(i + 2 < NCH)
            def _fi(i=i, b=b):
                fire_idx(i + 2, (b + 2) % 5)

            @pl.when(i + 1 < NCH)
            def _fg(b=b):
                wait_idx((b + 1) % 5)
                fire_gather((b + 1) % 5)

            wait_gather(b)
            fire_store(i, b)
        return carry

    lax.fori_loop(0, NCH // 5, step, 0)
    wait_store((NCH - 2) % 5)
    wait_store((NCH - 1) % 5)


def _embed(x2, table):
    return pl.kernel(
        _embed_body,
        out_type=(jax.ShapeDtypeStruct((N_PAD, 32), jnp.float32),
                  jax.ShapeDtypeStruct((N_PAD, 32), jnp.float32)),
        mesh=_MESH,
        compiler_params=_SC_PARAMS,
        scratch_types=[
            pltpu.VMEM_SHARED((2 * V, 32), jnp.float32),
            pltpu.VMEM((5, CE), jnp.int32),
            pltpu.VMEM((5, CE, 32), jnp.float32),
            pltpu.SemaphoreType.DMA((5,)),
            pltpu.SemaphoreType.DMA((5,)),
            pltpu.SemaphoreType.DMA((5,)),
        ],
    )(x2, table)


def _agg_body(h0, h1, src1, dst1, zrows, o0, o1, agg_s,
              sidx, didx, rows, semi, semg, sems):
    c = lax.axis_index("c")
    s = lax.axis_index("s")
    pltpu.sync_copy(zrows.at[pl.ds(s * ZERO_T, ZERO_T)],
                    agg_s.at[pl.ds(s * ZERO_T, ZERO_T)])
    plsc.subcore_barrier()

    ebase = s * E_T

    def fire_idx(t, slot):
        pltpu.async_copy(src1.at[pl.ds(ebase + t * CE, CE)], sidx.at[slot],
                         semi.at[slot])
        pltpu.async_copy(dst1.at[pl.ds(ebase + t * CE, CE)], didx.at[slot],
                         semi.at[slot])

    def wait_idx(slot):
        pltpu.make_async_copy(dst1.at[pl.ds(0, CE)], sidx.at[slot],
                              semi.at[slot]).wait()
        pltpu.make_async_copy(dst1.at[pl.ds(0, CE)], didx.at[slot],
                              semi.at[slot]).wait()

    def fire_gather(islot, rslot):
        @pl.when(c == 0)
        def _g0():
            pltpu.async_copy(h0.at[sidx.at[islot]], rows.at[rslot],
                             semg.at[rslot])

        @pl.when(c == 1)
        def _g1():
            pltpu.async_copy(h1.at[sidx.at[islot]], rows.at[rslot],
                             semg.at[rslot])

    def wait_gather(rslot):
        pltpu.make_async_copy(h0.at[pl.ds(0, CE)], rows.at[rslot],
                              semg.at[rslot]).wait()

    def fire_scatter(islot, rslot):
        pltpu.async_copy(rows.at[rslot], agg_s.at[didx.at[islot]],
                         sems.at[rslot], add=True)

    def wait_scatter(rslot):
        pltpu.make_async_copy(h0.at[pl.ds(0, CE)], rows.at[rslot],
                              sems.at[rslot]).wait()

    for t in range(4):
        fire_idx(t, t)
    for t in range(2):
        wait_idx(t)
        fire_gather(t, t)

    def step(jj, carry):
        for b in range(8):
            i = jj * 8 + b

            @pl.when(i >= 2)
            def _free(b=b):
                wait_scatter((b + 2) % 4)

            @pl.when(i + 4 < EDGE_C)
            def _idx(i=i, b=b):
                fire_idx(i + 4, (b + 4) % 8)

            @pl.when(i + 2 < EDGE_C)
            def _gath(b=b):
                wait_idx((b + 2) % 8)
                fire_gather((b + 2) % 8, (b + 2) % 4)

            wait_gather(b % 4)
            fire_scatter(b % 8, b % 4)
        return carry

    lax.fori_loop(0, EDGE_C // 8, step, 0)
    for t in range(EDGE_C - 2, EDGE_C):
        wait_scatter(t % 4)
    plsc.subcore_barrier()

    @pl.when(c == 0)
    def _o0():
        pltpu.sync_copy(agg_s.at[pl.ds(s * NODE_T, NODE_T)],
                        o0.at[pl.ds(s * NODE_T, NODE_T)])

    @pl.when(c == 1)
    def _o1():
        pltpu.sync_copy(agg_s.at[pl.ds(s * NODE_T, NODE_T)],
                        o1.at[pl.ds(s * NODE_T, NODE_T)])


def _agg(h0, h1, src1, dst1, zrows):
    return pl.kernel(
        _agg_body,
        out_type=(jax.ShapeDtypeStruct((N_PAD, 32), jnp.float32),
                  jax.ShapeDtypeStruct((N_PAD, 32), jnp.float32)),
        mesh=_MESH,
        compiler_params=_SC_PARAMS,
        scratch_types=[
            pltpu.VMEM_SHARED((AGG_ROWS, 32), jnp.float32),
            pltpu.VMEM((8, CE), jnp.int32),
            pltpu.VMEM((8, CE), jnp.int32),
            pltpu.VMEM((4, CE, 32), jnp.float32),
            pltpu.SemaphoreType.DMA((8,)),
            pltpu.SemaphoreType.DMA((4,)),
            pltpu.SemaphoreType.DMA((4,)),
        ],
    )(h0, h1, src1, dst1, zrows)


def _pool_body(h0, h1, batchp, ones_h, zpool, zcnt, outp, outc,
               pool_s, cnt_s, bidx, rows_v, ones_v, semr, semb):
    c = lax.axis_index("c")
    s = lax.axis_index("s")
    nbase = s * NODE_T
    NCH = NODE_T // CE

    @pl.when(s == 0)
    def _z():
        pltpu.sync_copy(zpool, pool_s)
        pltpu.sync_copy(zcnt, cnt_s)

    pltpu.sync_copy(ones_h, ones_v)
    plsc.subcore_barrier()

    def fire_read(t, slot):
        @pl.when(c == 0)
        def _r0():
            pltpu.async_copy(h0.at[pl.ds(nbase + t * CE, CE)],
                             rows_v.at[slot], semr.at[slot])

        @pl.when(c == 1)
        def _r1():
            pltpu.async_copy(h1.at[pl.ds(nbase + t * CE, CE)],
                             rows_v.at[slot], semr.at[slot])

        pltpu.async_copy(batchp.at[pl.ds(nbase + t * CE, CE)],
                         bidx.at[slot], semb.at[slot])

    def wait_read(slot):
        pltpu.make_async_copy(h0.at[pl.ds(0, CE)], rows_v.at[slot],
                              semr.at[slot]).wait()
        pltpu.make_async_copy(batchp.at[pl.ds(0, CE)], bidx.at[slot],
                              semb.at[slot]).wait()

    fire_read(0, 0)
    fire_read(1, 1)

    def step(jj, carry):
        for b in range(5):
            i = jj * 5 + b

            @pl.when(i + 2 < NCH)
            def _fr(i=i, b=b):
                fire_read(i + 2, (b + 2) % 5)

            wait_read(b)
            pltpu.sync_copy(rows_v.at[b], pool_s.at[bidx.at[b]], add=True)
            pltpu.sync_copy(ones_v, cnt_s.at[bidx.at[b]], add=True)
        return carry

    lax.fori_loop(0, NCH // 5, step, 0)
    plsc.subcore_barrier()

    @pl.when(s == 0)
    def _out():
        pltpu.sync_copy(pool_s, outp.at[c])
        pltpu.sync_copy(cnt_s, outc.at[c])


def _pool(h0, h1, batchp, ones_h, zpool, zcnt):
    return pl.kernel(
        _pool_body,
        out_type=(jax.ShapeDtypeStruct((NC, PB, 32), jnp.float32),
                  jax.ShapeDtypeStruct((NC, PB, 16), jnp.float32)),
        mesh=_MESH,
        compiler_params=_SC_PARAMS,
        scratch_types=[
            pltpu.VMEM_SHARED((PB, 32), jnp.float32),
            pltpu.VMEM_SHARED((PB, 16), jnp.float32),
            pltpu.VMEM((5, CE), jnp.int32),
            pltpu.VMEM((5, CE, 32), jnp.float32),
            pltpu.VMEM((CE, 16), jnp.float32),
            pltpu.SemaphoreType.DMA((5,)),
            pltpu.SemaphoreType.DMA((5,)),
        ],
    )(h0, h1, batchp, ones_h, zpool, zcnt)


def _mlp_body(eps_ref, h0_ref, h1_ref, a0_ref, a1_ref, w1a_ref, w1b_ref,
              b1_ref, w20_ref, w21_ref, b20_ref, b21_ref, o0_ref, o1_ref):
    scale = 1.0 + eps_ref[0]
    z0 = scale * h0_ref[...] + a0_ref[...]
    z1 = scale * h1_ref[...] + a1_ref[...]
    t = jnp.dot(z0, w1a_ref[...], preferred_element_type=jnp.float32)
    t += jnp.dot(z1, w1b_ref[...], preferred_element_type=jnp.float32)
    t = jnp.maximum(t + b1_ref[...], 0.0)
    y0 = jnp.dot(t, w20_ref[...], preferred_element_type=jnp.float32)
    y1 = jnp.dot(t, w21_ref[...], preferred_element_type=jnp.float32)
    o0_ref[...] = jnp.maximum(y0 + b20_ref[...], 0.0)
    o1_ref[...] = jnp.maximum(y1 + b21_ref[...], 0.0)


def _mlp_layer(eps_l, h0, h1, a0, a1, w1, b1, w2, b2):
    h0q = h0.reshape(MQ, 128)
    h1q = h1.reshape(MQ, 128)
    a0q = a0.reshape(MQ, 128)
    a1q = a1.reshape(MQ, 128)
    eye4 = jnp.eye(4, dtype=jnp.float32)
    w1a = jnp.kron(eye4, w1[:32, :])
    w1b = jnp.kron(eye4, w1[32:, :])
    b1big = jnp.tile(b1, 4)
    w20 = jnp.kron(eye4, w2[:, :32])
    w21 = jnp.kron(eye4, w2[:, 32:])
    b20 = jnp.tile(b2[:32], 4)
    b21 = jnp.tile(b2[32:], 4)
    grid = (MQ // RB,)
    blk = lambda i: (i, 0)
    fixed = lambda i: (0, 0)
    vec = lambda i: (0,)
    o0q, o1q = pl.pallas_call(
        _mlp_body,
        grid=grid,
        in_specs=[
            pl.BlockSpec(memory_space=pltpu.SMEM),
            pl.BlockSpec((RB, 128), blk),
            pl.BlockSpec((RB, 128), blk),
            pl.BlockSpec((RB, 128), blk),
            pl.BlockSpec((RB, 128), blk),
            pl.BlockSpec((128, 256), fixed),
            pl.BlockSpec((128, 256), fixed),
            pl.BlockSpec((256,), vec),
            pl.BlockSpec((256, 128), fixed),
            pl.BlockSpec((256, 128), fixed),
            pl.BlockSpec((128,), vec),
            pl.BlockSpec((128,), vec),
        ],
        out_specs=[pl.BlockSpec((RB, 128), blk), pl.BlockSpec((RB, 128), blk)],
        out_shape=[jax.ShapeDtypeStruct((MQ, 128), jnp.float32),
                   jax.ShapeDtypeStruct((MQ, 128), jnp.float32)],
    )(eps_l.reshape(1), h0q, h1q, a0q, a1q, w1a, w1b, b1big, w20, w21,
      b20, b21)
    return o0q.reshape(N_PAD, 32), o1q.reshape(N_PAD, 32)


def _head_body(p0_ref, p1_ref, cnt_ref, w1_ref, b1_ref, w2f_ref, sel_ref,
               b2_ref, out_ref):
    cnt = jnp.maximum(cnt_ref[...][:, 0:1], 1.0)
    g = jnp.concatenate([p0_ref[...], p1_ref[...]], axis=1) / cnt
    t = jnp.maximum(jnp.dot(g, w1_ref[...], preferred_element_type=jnp.float32)
                    + b1_ref[...], 0.0)
    u = t * w2f_ref[...]
    out_ref[...] = jnp.dot(u, sel_ref[...],
                           preferred_element_type=jnp.float32) + b2_ref[...]


def _head(p0, p1, cnt, w1cat, b1cat, w2flat, sel, b2row):
    P = sel.shape[1]
    return pl.pallas_call(
        _head_body,
        out_shape=jax.ShapeDtypeStruct((B, P), jnp.float32),
    )(p0, p1, cnt, w1cat, b1cat, w2flat, sel, b2row)


def kernel(x, edge_index, batch, embed, CW1, Cb1, CW2, Cb2, eps, HW1, Hb1,
           HW2, Hb2):
    x = x.astype(jnp.int32)
    src = edge_index[0].astype(jnp.int32)
    dst = edge_index[1].astype(jnp.int32)
    batch = batch.astype(jnp.int32)

    x_pad = jnp.pad(x, (0, N_PAD - N))
    x2 = jnp.concatenate([x_pad, x_pad + V])
    table = jnp.concatenate([embed[:, :32], embed[:, 32:]], axis=0)

    src1 = jnp.pad(src, (0, E_PAD - E))
    dst1 = jnp.pad(dst, (0, E_PAD - E), constant_values=N_PAD)
    zrows = jnp.zeros((AGG_ROWS, 32), jnp.float32)

    batchp = jnp.pad(batch, (0, N_PAD - N), constant_values=B)
    ones_h = jnp.ones((CE, 16), jnp.float32)
    zpool = jnp.zeros((PB, 32), jnp.float32)
    zcnt = jnp.zeros((PB, 16), jnp.float32)

    h0, h1 = _embed(x2, table)
    for l in range(3):
        a0, a1 = _agg(h0, h1, src1, dst1, zrows)
        h0, h1 = _mlp_layer(eps[l], h0, h1, a0, a1, CW1[l], Cb1[l], CW2[l],
                            Cb2[l])

    pool, cnt = _pool(h0, h1, batchp, ones_h, zpool, zcnt)

    P = HW1.shape[0]
    w1cat = HW1.transpose(1, 0, 2).reshape(H, P * H)
    b1cat = Hb1.reshape(P * H)
    w2flat = HW2[:, :, 0].reshape(P * H)
    sel = jnp.repeat(jnp.eye(P, dtype=jnp.float32), H, axis=0)
    b2row = Hb2[:, 0]
    return _head(pool[0, :B], pool[1, :B], cnt[0, :B], w1cat, b1cat, w2flat,
                 sel, b2row)
